# baseline (speedup 1.0000x reference)
import functools

import jax
import jax.numpy as jnp
from jax import lax
from jax.experimental import pallas as pl
from jax.experimental.pallas import tpu as pltpu
from jax.experimental.pallas import tpu_sc as plsc

_NUM_CORES = 2
_NUM_SUBCORES = 16
_NUM_WORKERS = _NUM_CORES * _NUM_SUBCORES
_CHUNK = 128
_LANES = 128
_MAX_RES = 256


def _down_body(emb_ref, wd_ref, gate_ref, h_ref):
    g = jax.nn.sigmoid(gate_ref[0])
    t = emb_ref[...]
    h_pre = jnp.dot(t, wd_ref[...], preferred_element_type=jnp.float32)
    h = (h_pre * jax.nn.sigmoid(h_pre)) * g
    pad = jnp.zeros((h.shape[0], _LANES - h.shape[1]), jnp.float32)
    h_ref[...] = jnp.concatenate([h, pad], axis=1)


def _up_body(bneck, h_ref, wu_ref, out_ref):
    h = h_ref[...][:, :bneck]
    out_ref[...] = jnp.dot(h, wu_ref[...], preferred_element_type=jnp.float32)


def _make_gather(bneck_pad, n_tokens):
    b_per_w = n_tokens // _NUM_WORKERS
    resident = min(_MAX_RES, b_per_w)
    n_bufs = b_per_w // resident
    n_chunks = resident // _CHUNK
    mesh = plsc.VectorSubcoreMesh(
        core_axis_name="c", subcore_axis_name="s",
        num_cores=_NUM_CORES, num_subcores=_NUM_SUBCORES)

    @functools.partial(
        pl.kernel,
        out_type=jax.ShapeDtypeStruct((n_tokens, bneck_pad), jnp.float32),
        mesh=mesh,
        scratch_types=[
            pltpu.VMEM((b_per_w,), jnp.int32),
            [pltpu.VMEM((resident, bneck_pad), jnp.float32)] * 2,
            pltpu.SemaphoreType.DMA,
            pltpu.SemaphoreType.DMA,
        ],
    )
    def gather_kernel(table_hbm, idx_hbm, out_hbm, idx_v, rows_bufs, sem_g,
                      sem_w):
        wid = lax.axis_index("s") * _NUM_CORES + lax.axis_index("c")
        base = wid * b_per_w
        pltpu.sync_copy(idx_hbm.at[pl.ds(base, b_per_w)], idx_v)
        writebacks = []
        for hh in range(n_bufs):
            rows_v = rows_bufs[hh % 2]
            copies = []
            for c in range(n_chunks):
                off = hh * resident + c * _CHUNK
                copies.append(pltpu.async_copy(
                    table_hbm.at[idx_v.at[pl.ds(off, _CHUNK)]],
                    rows_v.at[pl.ds(c * _CHUNK, _CHUNK)],
                    sem_g))
            for cp in copies:
                cp.wait()
            writebacks.append(pltpu.async_copy(
                rows_v, out_hbm.at[pl.ds(base + hh * resident, resident)],
                sem_w))
        for wb in writebacks:
            wb.wait()

    return gather_kernel


def kernel(teacher_emb, W_down, W_up, gate, token_ids):
    vocab, t_dim = teacher_emb.shape
    bneck = W_down.shape[0]
    m_dim = W_up.shape[0]
    b, s = token_ids.shape
    n_tokens = b * s

    wd_t = W_down.T
    wu_t = W_up.T

    vb = 4096
    h_table = pl.pallas_call(
        _down_body,
        grid=(pl.cdiv(vocab, vb),),
        in_specs=[
            pl.BlockSpec((vb, t_dim), lambda i: (i, 0)),
            pl.BlockSpec((t_dim, bneck), lambda i: (0, 0)),
            pl.BlockSpec(memory_space=pltpu.SMEM),
        ],
        out_specs=pl.BlockSpec((vb, _LANES), lambda i: (i, 0)),
        out_shape=jax.ShapeDtypeStruct((vocab, _LANES), jnp.float32),
    )(teacher_emb, wd_t, gate)

    ids_flat = token_ids.reshape(n_tokens)
    h_tok = _make_gather(_LANES, n_tokens)(h_table, ids_flat)

    tb = 2048
    out_flat = pl.pallas_call(
        functools.partial(_up_body, bneck),
        grid=(n_tokens // tb,),
        in_specs=[
            pl.BlockSpec((tb, _LANES), lambda i: (i, 0)),
            pl.BlockSpec((bneck, m_dim), lambda i: (0, 0)),
        ],
        out_specs=pl.BlockSpec((tb, m_dim), lambda i: (i, 0)),
        out_shape=jax.ShapeDtypeStruct((n_tokens, m_dim), jnp.float32),
    )(h_tok, wu_t)

    return out_flat.reshape(b, s, m_dim)


# speedup vs baseline: 1.0183x; 1.0183x over previous
import functools

import jax
import jax.numpy as jnp
from jax import lax
from jax.experimental import pallas as pl
from jax.experimental.pallas import tpu as pltpu
from jax.experimental.pallas import tpu_sc as plsc

_NUM_CORES = 2
_NUM_SUBCORES = 16
_NUM_WORKERS = _NUM_CORES * _NUM_SUBCORES
_CHUNK = 128
_LANES = 128
_MAX_RES = 448


def _down_body(emb_ref, wd_ref, gate_ref, h_ref):
    g = jax.nn.sigmoid(gate_ref[0])
    t = emb_ref[...]
    h_pre = jnp.dot(t, wd_ref[...], preferred_element_type=jnp.float32)
    h = (h_pre * jax.nn.sigmoid(h_pre)) * g
    pad = jnp.zeros((h.shape[0], _LANES - h.shape[1]), jnp.float32)
    h_ref[...] = jnp.concatenate([h, pad], axis=1)


def _up_body(bneck, h_ref, wu_ref, out_ref):
    h = h_ref[...][:, :bneck]
    out_ref[...] = jnp.dot(h, wu_ref[...], preferred_element_type=jnp.float32)


def _split(total, piece):
    sizes = []
    while total > 0:
        sizes.append(min(piece, total))
        total -= sizes[-1]
    return sizes


def _make_gather(bneck_pad, n_tokens):
    b_per_w = n_tokens // _NUM_WORKERS
    pieces = _split(b_per_w, _MAX_RES)
    offs = [sum(pieces[:i]) for i in range(len(pieces))]
    slot_rows = min(_MAX_RES, b_per_w)
    mesh = plsc.VectorSubcoreMesh(
        core_axis_name="c", subcore_axis_name="s",
        num_cores=_NUM_CORES, num_subcores=_NUM_SUBCORES)

    @functools.partial(
        pl.kernel,
        out_type=jax.ShapeDtypeStruct((n_tokens, bneck_pad), jnp.float32),
        mesh=mesh,
        scratch_types=[
            pltpu.VMEM((b_per_w,), jnp.int32),
            [pltpu.VMEM((slot_rows, bneck_pad), jnp.float32)] * 2,
            [pltpu.SemaphoreType.DMA] * 2,
            [pltpu.SemaphoreType.DMA] * 2,
        ],
    )
    def gather_kernel(table_hbm, idx_hbm, out_hbm, idx_v, slots, sems_g,
                      sems_w):
        wid = lax.axis_index("s") * _NUM_CORES + lax.axis_index("c")
        base = wid * b_per_w
        pltpu.sync_copy(idx_hbm.at[pl.ds(base, b_per_w)], idx_v)

        def fire(i):
            descs = []
            for c, csz in enumerate(_split(pieces[i], _CHUNK)):
                off = offs[i] + c * _CHUNK
                descs.append(pltpu.async_copy(
                    table_hbm.at[idx_v.at[pl.ds(off, csz)]],
                    slots[i % 2].at[pl.ds(c * _CHUNK, csz)],
                    sems_g[i % 2]))
            return descs

        in_flight = {0: fire(0)}
        if len(pieces) > 1:
            in_flight[1] = fire(1)
        writebacks = {}
        for i in range(len(pieces)):
            for cp in in_flight.pop(i):
                cp.wait()
            writebacks[i] = pltpu.async_copy(
                slots[i % 2].at[pl.ds(0, pieces[i])],
                out_hbm.at[pl.ds(base + offs[i], pieces[i])],
                sems_w[i % 2])
            j = i + 2
            if j < len(pieces):
                writebacks.pop(i).wait()
                in_flight[j] = fire(j)
        for wb in writebacks.values():
            wb.wait()

    return gather_kernel


def kernel(teacher_emb, W_down, W_up, gate, token_ids):
    vocab, t_dim = teacher_emb.shape
    bneck = W_down.shape[0]
    m_dim = W_up.shape[0]
    b, s = token_ids.shape
    n_tokens = b * s

    wd_t = W_down.T
    wu_t = W_up.T

    vb = 4096
    h_table = pl.pallas_call(
        _down_body,
        grid=(pl.cdiv(vocab, vb),),
        in_specs=[
            pl.BlockSpec((vb, t_dim), lambda i: (i, 0)),
            pl.BlockSpec((t_dim, bneck), lambda i: (0, 0)),
            pl.BlockSpec(memory_space=pltpu.SMEM),
        ],
        out_specs=pl.BlockSpec((vb, _LANES), lambda i: (i, 0)),
        out_shape=jax.ShapeDtypeStruct((vocab, _LANES), jnp.float32),
    )(teacher_emb, wd_t, gate)

    ids_flat = token_ids.reshape(n_tokens)
    h_tok = _make_gather(_LANES, n_tokens)(h_table, ids_flat)

    tb = 2048
    out_flat = pl.pallas_call(
        functools.partial(_up_body, bneck),
        grid=(n_tokens // tb,),
        in_specs=[
            pl.BlockSpec((tb, _LANES), lambda i: (i, 0)),
            pl.BlockSpec((bneck, m_dim), lambda i: (0, 0)),
        ],
        out_specs=pl.BlockSpec((tb, m_dim), lambda i: (i, 0)),
        out_shape=jax.ShapeDtypeStruct((n_tokens, m_dim), jnp.float32),
    )(h_tok, wu_t)

    return out_flat.reshape(b, s, m_dim)
